# Initial kernel scaffold; baseline (speedup 1.0000x reference)
import jax
import jax.numpy as jnp
from jax.experimental import pallas as pl


def kernel(x, table):
    raise NotImplementedError("write your pallas kernel here")


# speedup vs baseline: 2.9953x; 2.9953x over previous
import functools

import jax
import jax.numpy as jnp
from jax import lax
from jax.experimental import pallas as pl
from jax.experimental.pallas import tpu as pltpu
from jax.experimental.pallas import tpu_sc as plsc

CH = 128


def _make_emb(N, V, D, NC, NS):
    NW = NC * NS
    n_per_w = N // NW
    G = n_per_w // CH
    mesh = plsc.VectorSubcoreMesh(core_axis_name="c", subcore_axis_name="s")

    @functools.partial(
        pl.kernel,
        mesh=mesh,
        out_type=jax.ShapeDtypeStruct((N, D), jnp.float32),
        scratch_types=[
            pltpu.VMEM((n_per_w,), jnp.int32),
            pltpu.VMEM((CH, D), jnp.float32),
            pltpu.SemaphoreType.DMA,
        ],
    )
    def emb(table_hbm, idx_hbm, out_hbm, idx_v, rows_v, sem):
        wid = lax.axis_index("s") * NC + lax.axis_index("c")
        base = wid * n_per_w
        pltpu.sync_copy(idx_hbm.at[pl.ds(base, n_per_w)], idx_v)

        def body(g, carry):
            off = pl.multiple_of(g * CH, CH)
            pltpu.async_copy(
                table_hbm.at[idx_v.at[pl.ds(off, CH)]], rows_v, sem
            ).wait()
            pltpu.sync_copy(rows_v, out_hbm.at[pl.ds(base + off, CH)])
            return carry

        lax.fori_loop(0, G, body, 0)

    return emb


def kernel(x, table):
    B, F = x.shape
    V, D = table.shape
    N = B * F
    info = plsc.get_sparse_core_info()
    emb = _make_emb(N, V, D, info.num_cores, info.num_subcores)
    out = emb(table, x.reshape(N).astype(jnp.int32))
    return out.reshape(B, F, D)


# speedup vs baseline: 3.3902x; 1.1319x over previous
import functools

import jax
import jax.numpy as jnp
from jax import lax
from jax.experimental import pallas as pl
from jax.experimental.pallas import tpu as pltpu
from jax.experimental.pallas import tpu_sc as plsc

CH = 128
K = 2


def _make_emb(N, V, D, NC, NS):
    NW = NC * NS
    n_per_w = N // NW
    GR = K * CH
    G = n_per_w // GR
    mesh = plsc.VectorSubcoreMesh(core_axis_name="c", subcore_axis_name="s")

    @functools.partial(
        pl.kernel,
        mesh=mesh,
        out_type=jax.ShapeDtypeStruct((N, D), jnp.float32),
        scratch_types=[
            pltpu.VMEM((n_per_w,), jnp.int32),
            pltpu.VMEM((2 * GR, D), jnp.float32),
            pltpu.SemaphoreType.DMA,
            pltpu.SemaphoreType.DMA,
        ],
    )
    def emb(table_hbm, idx_hbm, out_hbm, idx_v, bufs, sem_a, sem_b):
        wid = lax.axis_index("s") * NC + lax.axis_index("c")
        base = wid * n_per_w
        pltpu.sync_copy(idx_hbm.at[pl.ds(base, n_per_w)], idx_v)

        def fire(g, half, sem):
            for k in range(K):
                off = pl.multiple_of(g * GR + k * CH, CH)
                pltpu.async_copy(
                    table_hbm.at[idx_v.at[pl.ds(off, CH)]],
                    bufs.at[pl.ds(half * GR + k * CH, CH)],
                    sem,
                )

        def drain(half, sem):
            pltpu.make_async_copy(
                table_hbm.at[pl.ds(0, GR)],
                bufs.at[pl.ds(half * GR, GR)],
                sem,
            ).wait()

        def out_copy(g, half):
            pltpu.sync_copy(
                bufs.at[pl.ds(half * GR, GR)],
                out_hbm.at[pl.ds(base + g * GR, GR)],
            )

        fire(0, 0, sem_a)

        def body(i, carry):
            g0 = i * 2
            fire(g0 + 1, 1, sem_b)
            drain(0, sem_a)
            out_copy(g0, 0)

            @pl.when(g0 + 2 < G)
            def _():
                fire(g0 + 2, 0, sem_a)

            drain(1, sem_b)
            out_copy(g0 + 1, 1)
            return carry

        lax.fori_loop(0, G // 2, body, 0)

    return emb


def kernel(x, table):
    B, F = x.shape
    V, D = table.shape
    N = B * F
    info = plsc.get_sparse_core_info()
    emb = _make_emb(N, V, D, info.num_cores, info.num_subcores)
    out = emb(table, x.reshape(N).astype(jnp.int32))
    return out.reshape(B, F, D)


# speedup vs baseline: 5.7956x; 1.7095x over previous
import functools

import jax
import jax.numpy as jnp
from jax import lax
from jax.experimental import pallas as pl
from jax.experimental.pallas import tpu as pltpu
from jax.experimental.pallas import tpu_sc as plsc

GB = 8


def _make_emb(B, F, V, D, NC, NS):
    NW = NC * NS
    b_per_w = B // NW
    G = b_per_w // GB
    mesh = plsc.VectorSubcoreMesh(core_axis_name="c", subcore_axis_name="s")

    @functools.partial(
        pl.kernel,
        mesh=mesh,
        out_type=jax.ShapeDtypeStruct((B, F, D), jnp.float32),
        scratch_types=[
            pltpu.VMEM((b_per_w, F), jnp.int32),
            pltpu.VMEM((2, GB, F, D), jnp.float32),
            pltpu.SemaphoreType.DMA,
            pltpu.SemaphoreType.DMA,
        ],
    )
    def emb(table_hbm, idx_hbm, out_hbm, idx_v, bufs, sem_a, sem_b):
        wid = lax.axis_index("s") * NC + lax.axis_index("c")
        base = wid * b_per_w
        pltpu.sync_copy(idx_hbm.at[pl.ds(base, b_per_w)], idx_v)

        def fire(g, half, sem):
            for j in range(GB):
                pltpu.async_copy(
                    table_hbm.at[idx_v.at[g * GB + j]],
                    bufs.at[half, j],
                    sem,
                )

        def drain(g, half, sem):
            pltpu.make_async_copy(
                out_hbm.at[pl.ds(base + g * GB, GB)],
                bufs.at[half],
                sem,
            ).wait()

        def out_copy(g, half):
            pltpu.sync_copy(
                bufs.at[half],
                out_hbm.at[pl.ds(base + g * GB, GB)],
            )

        fire(0, 0, sem_a)

        def body(i, carry):
            g0 = i * 2
            fire(g0 + 1, 1, sem_b)
            drain(g0, 0, sem_a)
            out_copy(g0, 0)

            @pl.when(g0 + 2 < G)
            def _():
                fire(g0 + 2, 0, sem_a)

            drain(g0 + 1, 1, sem_b)
            out_copy(g0 + 1, 1)
            return carry

        lax.fori_loop(0, G // 2, body, 0)

    return emb


def kernel(x, table):
    B, F = x.shape
    V, D = table.shape
    info = plsc.get_sparse_core_info()
    emb = _make_emb(B, F, V, D, info.num_cores, info.num_subcores)
    return emb(table, x.astype(jnp.int32))


# speedup vs baseline: 11.9297x; 2.0584x over previous
import functools

import jax
import jax.numpy as jnp
from jax import lax
from jax.experimental import pallas as pl
from jax.experimental.pallas import tpu as pltpu
from jax.experimental.pallas import tpu_sc as plsc

CH = 128
K = 2


def _make_emb(N, V, D, NC, NS):
    NW = NC * NS
    n_per_w = N // NW
    GR = K * CH
    G = n_per_w // GR
    mesh = plsc.VectorSubcoreMesh(core_axis_name="c", subcore_axis_name="s")

    @functools.partial(
        pl.kernel,
        mesh=mesh,
        out_type=jax.ShapeDtypeStruct((N, D), jnp.float32),
        scratch_types=[
            pltpu.VMEM((n_per_w,), jnp.int32),
            pltpu.VMEM((2 * GR, D), jnp.float32),
            pltpu.SemaphoreType.DMA,
            pltpu.SemaphoreType.DMA,
        ],
    )
    def emb(table_hbm, idx_hbm, out_hbm, idx_v, bufs, sem_a, sem_b):
        wid = lax.axis_index("s") * NC + lax.axis_index("c")
        base = wid * n_per_w
        pltpu.sync_copy(idx_hbm.at[pl.ds(base, n_per_w)], idx_v)

        def fire(g, half, sem):
            for k in range(K):
                off = pl.multiple_of(g * GR + k * CH, CH)
                pltpu.async_copy(
                    table_hbm.at[idx_v.at[pl.ds(off, CH)]],
                    bufs.at[pl.ds(half * GR + k * CH, CH)],
                    sem,
                )

        def drain(half, sem):
            pltpu.make_async_copy(
                table_hbm.at[pl.ds(0, GR)],
                bufs.at[pl.ds(half * GR, GR)],
                sem,
            ).wait()

        def out_copy(g, half):
            pltpu.sync_copy(
                bufs.at[pl.ds(half * GR, GR)],
                out_hbm.at[pl.ds(base + g * GR, GR)],
            )

        fire(0, 0, sem_a)

        def body(i, carry):
            g0 = i * 2
            fire(g0 + 1, 1, sem_b)
            drain(0, sem_a)
            out_copy(g0, 0)

            @pl.when(g0 + 2 < G)
            def _():
                fire(g0 + 2, 0, sem_a)

            drain(1, sem_b)
            out_copy(g0 + 1, 1)
            return carry

        lax.fori_loop(0, G // 2, body, 0)

    return emb


def kernel(x, table):
    B, F = x.shape
    V, D = table.shape
    N = B * F
    info = plsc.get_sparse_core_info()
    emb = _make_emb(N, V, D, info.num_cores, info.num_subcores)
    idx_fmajor = jnp.transpose(x).reshape(N).astype(jnp.int32)
    out = emb(table, idx_fmajor)
    return out.reshape(F, B, D).transpose(1, 0, 2)
